# baseline (speedup 1.0000x reference)
import functools

import jax
import jax.numpy as jnp
from jax import lax
from jax.experimental import pallas as pl
from jax.experimental.pallas import tpu as pltpu
from jax.experimental.pallas import tpu_sc as plsc

_VOCAB = 1000000
_EMBED = 64
_BATCH = 4096
_SEQ = 200

_N = _BATCH * _SEQ
_NW = 32
_ROWS_PER_W = _N // _NW
_CHUNK = 128
_NCHUNK = _ROWS_PER_W // _CHUNK
_NBUF = 5
_NLAP = _NCHUNK // _NBUF

_mesh = plsc.VectorSubcoreMesh(core_axis_name="c", subcore_axis_name="s")


@functools.partial(
    pl.kernel,
    mesh=_mesh,
    out_type=jax.ShapeDtypeStruct((_SEQ, _BATCH, 2 * _EMBED), jnp.float32),
    scratch_types=[
        pltpu.VMEM((_NCHUNK, _CHUNK), jnp.int32),
        pltpu.VMEM((_NBUF, _CHUNK, 2 * _EMBED), jnp.float32),
        pltpu.SemaphoreType.DMA,
    ],
    compiler_params=pltpu.CompilerParams(use_tc_tiling_on_sc=True),
)
def _gather_rows(idx_hbm, table_hbm, out_hbm, idx_v, rows_v, sem):
    wid = lax.axis_index("s") * 2 + lax.axis_index("c")
    base = wid * _ROWS_PER_W
    pltpu.sync_copy(idx_hbm.at[pl.ds(wid * _NCHUNK, _NCHUNK)], idx_v)

    def gather(j, b):
        pltpu.async_copy(table_hbm.at[idx_v.at[j]], rows_v.at[b], sem)

    def wait_rows(b):
        pltpu.make_async_copy(
            table_hbm.at[pl.ds(0, _CHUNK)], rows_v.at[b], sem
        ).wait()

    def store(b, j):
        r0 = base + j * _CHUNK
        s = r0 // _BATCH
        b0 = lax.rem(r0, _BATCH)
        pltpu.sync_copy(rows_v.at[b], out_hbm.at[s, pl.ds(b0, _CHUNK)])

    for b in range(_NBUF):
        gather(b, b)

    def lap(g, carry):
        for b in range(_NBUF):
            j = g * _NBUF + b
            wait_rows(b)
            store(b, j)
            gather(j + _NBUF, b)
        return carry

    lax.fori_loop(0, _NLAP - 1, lap, 0)

    for b in range(_NBUF):
        j = (_NLAP - 1) * _NBUF + b
        wait_rows(b)
        store(b, j)


_TBLK = 8192
_TGRID = (_VOCAB + _TBLK - 1) // _TBLK


def _tc_widen_body(in_ref, out_ref):
    out_ref[:, : _EMBED] = in_ref[...].T


_tc_widen = pl.pallas_call(
    _tc_widen_body,
    grid=(_TGRID,),
    in_specs=[pl.BlockSpec((_EMBED, _TBLK), lambda j: (0, j))],
    out_specs=pl.BlockSpec((_TBLK, 2 * _EMBED), lambda j: (j, 0)),
    out_shape=jax.ShapeDtypeStruct((_VOCAB, 2 * _EMBED), jnp.float32),
)


def kernel(inputs, table):
    idx = inputs.T.reshape(_N // _CHUNK, _CHUNK).astype(jnp.int32)
    table128 = _tc_widen(table.T)
    return _gather_rows(idx, table128)[:, :, :_EMBED]


# speedup vs baseline: 1.0283x; 1.0283x over previous
import functools

import jax
import jax.numpy as jnp
from jax import lax
from jax.experimental import pallas as pl
from jax.experimental.pallas import tpu as pltpu
from jax.experimental.pallas import tpu_sc as plsc

_VOCAB = 1000000
_EMBED = 64
_BATCH = 4096
_SEQ = 200

_N = _BATCH * _SEQ
_NW = 32
_ROWS_PER_W = _N // _NW
_CHUNK = 128
_NCHUNK = _ROWS_PER_W // _CHUNK
_NBUF = 5
_NLAP = _NCHUNK // _NBUF

_mesh = plsc.VectorSubcoreMesh(core_axis_name="c", subcore_axis_name="s")


@functools.partial(
    pl.kernel,
    mesh=_mesh,
    out_type=jax.ShapeDtypeStruct((_SEQ, _BATCH, 2 * _EMBED), jnp.float32),
    scratch_types=[
        pltpu.VMEM((_NCHUNK, _CHUNK), jnp.int32),
        pltpu.VMEM((_NBUF, _CHUNK, 2 * _EMBED), jnp.float32),
        pltpu.SemaphoreType.DMA,
    ],
    compiler_params=pltpu.CompilerParams(use_tc_tiling_on_sc=True),
)
def _gather_rows(idx_hbm, table_hbm, out_hbm, idx_v, rows_v, sem):
    wid = lax.axis_index("s") * 2 + lax.axis_index("c")
    base = wid * _ROWS_PER_W
    pltpu.sync_copy(idx_hbm.at[pl.ds(wid * _NCHUNK, _NCHUNK)], idx_v)

    def gather(j, b):
        pltpu.async_copy(table_hbm.at[idx_v.at[j]], rows_v.at[b], sem)

    def wait_rows(b):
        pltpu.make_async_copy(
            table_hbm.at[pl.ds(0, _CHUNK)], rows_v.at[b], sem
        ).wait()

    def store(b, j):
        r0 = base + j * _CHUNK
        s = r0 // _BATCH
        b0 = lax.rem(r0, _BATCH)
        pltpu.sync_copy(rows_v.at[b], out_hbm.at[s, pl.ds(b0, _CHUNK)])

    for b in range(_NBUF):
        gather(b, b)

    def lap(g, carry):
        for b in range(_NBUF):
            j = g * _NBUF + b
            wait_rows(b)
            store(b, j)
            gather(j + _NBUF, b)
        return carry

    lax.fori_loop(0, _NLAP - 1, lap, 0)

    for b in range(_NBUF):
        j = (_NLAP - 1) * _NBUF + b
        wait_rows(b)
        store(b, j)


_TBLK = 16384
_TGRID = (_VOCAB + _TBLK - 1) // _TBLK


def _tc_widen_body(in_ref, out_ref):
    out_ref[:, : _EMBED] = in_ref[...].T


_tc_widen = pl.pallas_call(
    _tc_widen_body,
    grid=(_TGRID,),
    in_specs=[pl.BlockSpec((_EMBED, _TBLK), lambda j: (0, j))],
    out_specs=pl.BlockSpec((_TBLK, 2 * _EMBED), lambda j: (j, 0)),
    out_shape=jax.ShapeDtypeStruct((_VOCAB, 2 * _EMBED), jnp.float32),
)


def kernel(inputs, table):
    idx = inputs.T.reshape(_N // _CHUNK, _CHUNK).astype(jnp.int32)
    table128 = _tc_widen(table.T)
    return _gather_rows(idx, table128)[:, :, :_EMBED]


# speedup vs baseline: 1.0401x; 1.0115x over previous
import functools

import jax
import jax.numpy as jnp
from jax import lax
from jax.experimental import pallas as pl
from jax.experimental.pallas import tpu as pltpu
from jax.experimental.pallas import tpu_sc as plsc

_VOCAB = 1000000
_EMBED = 64
_BATCH = 4096
_SEQ = 200

_N = _BATCH * _SEQ
_NW = 32
_ROWS_PER_W = _N // _NW
_CHUNK = 128
_NCHUNK = _ROWS_PER_W // _CHUNK
_NBUF = 5
_NLAP = _NCHUNK // _NBUF

_mesh = plsc.VectorSubcoreMesh(core_axis_name="c", subcore_axis_name="s")


@functools.partial(
    pl.kernel,
    mesh=_mesh,
    out_type=jax.ShapeDtypeStruct((_SEQ, _BATCH, 2 * _EMBED), jnp.float32),
    scratch_types=[
        pltpu.VMEM((_NCHUNK, _CHUNK), jnp.int32),
        pltpu.VMEM((_NBUF, _CHUNK, 2 * _EMBED), jnp.float32),
        pltpu.SemaphoreType.DMA,
    ],
    compiler_params=pltpu.CompilerParams(use_tc_tiling_on_sc=True),
)
def _gather_rows(idx_hbm, table_hbm, out_hbm, idx_v, rows_v, sem):
    wid = lax.axis_index("s") * 2 + lax.axis_index("c")
    base = wid * _ROWS_PER_W
    pltpu.sync_copy(idx_hbm.at[pl.ds(wid * _NCHUNK, _NCHUNK)], idx_v)

    def gather(j, b):
        pltpu.async_copy(table_hbm.at[idx_v.at[j]], rows_v.at[b], sem)

    def wait_rows(b):
        pltpu.make_async_copy(
            table_hbm.at[pl.ds(0, _CHUNK)], rows_v.at[b], sem
        ).wait()

    def store(b, j):
        r0 = base + j * _CHUNK
        s = r0 // _BATCH
        b0 = lax.rem(r0, _BATCH)
        pltpu.sync_copy(rows_v.at[b], out_hbm.at[s, pl.ds(b0, _CHUNK)])

    for b in range(_NBUF):
        gather(b, b)

    def lap(g, carry):
        for b in range(_NBUF):
            j = g * _NBUF + b
            wait_rows(b)
            store(b, j)
            gather(j + _NBUF, b)
        return carry

    lax.fori_loop(0, _NLAP - 1, lap, 0)

    for b in range(_NBUF):
        j = (_NLAP - 1) * _NBUF + b
        wait_rows(b)
        store(b, j)


_TBLK = 32768
_TGRID = (_VOCAB + _TBLK - 1) // _TBLK


def _tc_widen_body(in_ref, out_ref):
    out_ref[:, : _EMBED] = in_ref[...].T


_tc_widen = pl.pallas_call(
    _tc_widen_body,
    grid=(_TGRID,),
    in_specs=[pl.BlockSpec((_EMBED, _TBLK), lambda j: (0, j))],
    out_specs=pl.BlockSpec((_TBLK, 2 * _EMBED), lambda j: (j, 0)),
    out_shape=jax.ShapeDtypeStruct((_VOCAB, 2 * _EMBED), jnp.float32),
)


def kernel(inputs, table):
    idx = inputs.T.reshape(_N // _CHUNK, _CHUNK).astype(jnp.int32)
    table128 = _tc_widen(table.T)
    return _gather_rows(idx, table128)[:, :, :_EMBED]
